# baseline (speedup 1.0000x reference)
import dataclasses
import functools

import jax
import jax.numpy as jnp
import numpy as np
from jax import lax
from jax.experimental import pallas as pl
from jax.experimental.pallas import tpu as pltpu
from jax.experimental.pallas import tpu_sc as plsc

_INPUT_DIM = 3
_NUM_LEVELS = 16
_LEVEL_DIM = 2
_BASE_RES = 16
_MAX_PARAMS = 2 ** 19
_HASH_MASK = _MAX_PARAMS - 1
_PRIMES_I32 = tuple(int(np.uint32(p).astype(np.int64) - (1 << 32) if p >= 1 << 31 else p)
                    for p in (1, 2654435761, 805459861))


def _make_offsets():
    offs, o = [], 0
    for i in range(_NUM_LEVELS):
        res = _BASE_RES * 2 ** i
        offs.append(o)
        o += min(_MAX_PARAMS, (res + 1) ** _INPUT_DIM)
    offs.append(o)
    return offs


_OFFS = _make_offsets()
_NUM_LINEAR = 3

_NC, _NS = 2, 16
_NW = _NC * _NS
_C = 1024
_WIN = 128
_NWIN = 8 * _C // _WIN


def _sc_body(xT, emb, out, xyz_v, frac_v, idx_v, rows_v, out_v, gsem):
    wid = lax.axis_index("s") * _NC + lax.axis_index("c")
    B = out.shape[0]
    ppw = B // _NW

    iota = lax.iota(jnp.int32, 16)
    zeros_i = jnp.zeros((16,), jnp.int32)
    ones_i = zeros_i + 1

    def idx_pass(scale_f, mults, use_xor, use_mask, offset):
        @pl.loop(0, _C, step=16)
        def _(p):
            a, b = [], []
            for d in range(3):
                c = xyz_v[pl.ds(d * _C + p, 16)]
                x = (c + 1.0) * 0.5
                pos = x * scale_f + 0.5
                gi = pos.astype(jnp.int32)
                frac_v[pl.ds(d * _C + p, 16)] = pos - gi.astype(jnp.float32)
                m = mults[d]
                a.append(gi * m if m != 1 else gi)
                b.append((gi + 1) * m if m != 1 else gi + 1)
            for k in range(8):
                t0 = b[0] if k & 1 else a[0]
                t1 = b[1] if k & 2 else a[1]
                t2 = b[2] if k & 4 else a[2]
                if use_xor:
                    h = (t0 ^ t1) ^ t2
                else:
                    h = (t0 + t1) + t2
                if use_mask:
                    h = h & _HASH_MASK
                idx_v[pl.ds(k * _C + p, 16)] = h + offset

    def gather():
        @pl.loop(0, _NWIN)
        def _(j):
            pltpu.async_copy(emb.at[idx_v.at[pl.ds(j * _WIN, _WIN)]],
                             rows_v.at[pl.ds(j * _WIN, _WIN)], gsem).wait()

    def acc_pass(col2):
        @pl.loop(0, _C, step=16)
        def _(p):
            f = [frac_v[pl.ds(d * _C + p, 16)] for d in range(3)]
            g = [1.0 - fd for fd in f]
            u = [g[0] * g[1], f[0] * g[1], g[0] * f[1], f[0] * f[1]]
            w = [u[k & 3] * (f[2] if k & 4 else g[2]) for k in range(8)]
            row = iota + p
            acc0 = jnp.zeros((16,), jnp.float32)
            acc1 = jnp.zeros((16,), jnp.float32)
            for k in range(8):
                ridx = row + (k * _C)
                acc0 = acc0 + w[k] * plsc.load_gather(rows_v, [ridx, zeros_i])
                acc1 = acc1 + w[k] * plsc.load_gather(rows_v, [ridx, ones_i])
            plsc.store_scatter(out_v, [row, zeros_i + col2], acc0)
            plsc.store_scatter(out_v, [row, zeros_i + (col2 + 1)], acc1)

    @pl.loop(0, ppw // _C)
    def _(ci):
        base = wid * ppw + ci * _C
        for d in range(3):
            pltpu.sync_copy(xT.at[pl.ds(d * B + base, _C)],
                            xyz_v.at[pl.ds(d * _C, _C)])

        for l in range(_NUM_LINEAR):
            res = _BASE_RES << l
            s = res + 1
            idx_pass(float(res - 1), (1, s, s * s), False, False, _OFFS[l])
            gather()
            acc_pass(2 * l)

        @pl.loop(_NUM_LINEAR, _NUM_LEVELS)
        def _(lv):
            res_i = jnp.left_shift(jnp.int32(_BASE_RES), lv)
            scale_f = (res_i - 1).astype(jnp.float32)
            offset = jnp.int32(_OFFS[_NUM_LINEAR] - _NUM_LINEAR * _MAX_PARAMS) \
                + lv * _MAX_PARAMS
            idx_pass(scale_f, _PRIMES_I32, True, True, offset)
            gather()
            acc_pass(2 * lv)

        pltpu.sync_copy(out_v, out.at[pl.ds(base, _C)])


def kernel(inputs, embeddings):
    B = inputs.shape[0]
    assert B % (_NW * _C) == 0
    xT = inputs.T.reshape(3 * B)
    mesh = plsc.VectorSubcoreMesh(core_axis_name="c", subcore_axis_name="s")
    cp = pltpu.CompilerParams(use_tc_tiling_on_sc=False)
    if "needs_layout_passes" in pltpu.CompilerParams.__dataclass_fields__:
        cp = dataclasses.replace(cp, needs_layout_passes=False)
    kfn = pl.kernel(
        _sc_body,
        out_type=jax.ShapeDtypeStruct((B, _NUM_LEVELS * _LEVEL_DIM), jnp.float32),
        mesh=mesh,
        scratch_types=[
            pltpu.VMEM((3 * _C,), jnp.float32),
            pltpu.VMEM((3 * _C,), jnp.float32),
            pltpu.VMEM((8 * _C,), jnp.int32),
            pltpu.VMEM((8 * _C, 2), jnp.float32),
            pltpu.VMEM((_C, 32), jnp.float32),
            pltpu.SemaphoreType.DMA,
        ],
        compiler_params=cp,
    )
    return kfn(xT, embeddings)
